# baseline (speedup 1.0000x reference)
import functools

import jax
import jax.numpy as jnp
from jax import lax
from jax.experimental import pallas as pl
from jax.experimental.pallas import tpu as pltpu
from jax.experimental.pallas import tpu_sc as plsc

N = 10000
E = 320000
C = 128
H = 64
W = 64
BE = 2560
NTILES = 16
PT = E // NTILES
CH = 32
NCHUNK = PT // CH
FH = 40
NFC = N // FH


def _ln(x, jmat, g, b, eps=1e-5):
    mu = jnp.dot(x, jmat, preferred_element_type=jnp.float32)
    m2 = jnp.dot(x * x, jmat, preferred_element_type=jnp.float32)
    var = m2 - mu * mu
    return (x - mu) * jax.lax.rsqrt(var + eps) * g + b


def _gelu(x):
    return x * 0.5 * (1.0 + lax.erf(x * (2.0 ** -0.5)))


def _rne16(u):
    lsb = jnp.bitwise_and(lax.shift_right_logical(u, 16), 1)
    return lax.shift_right_logical(u + 32767 + lsb, 16)


def _pack_pair(lo_f32, hi_f32):
    lo = _rne16(lax.bitcast_convert_type(lo_f32, jnp.int32))
    hi = _rne16(lax.bitcast_convert_type(hi_f32, jnp.int32))
    return jnp.bitwise_or(lax.shift_left(hi, 16), lo)


def _prep_body(hood, dw1t, db1, g1, b1, dw2t, db2, g2, b2, jm, d0):
    jmat = jm[...]
    h = jnp.dot(hood[...], dw1t[...], preferred_element_type=jnp.float32)
    h = h + db1[...]
    h = _ln(h, jmat, g1[...], b1[...])
    h = _gelu(h)
    h = jnp.dot(h, dw2t[...], preferred_element_type=jnp.float32) + db2[...]
    h = _ln(h, jmat, g2[...], b2[...])
    d = _gelu(h)
    dr = d.reshape(BE // 2, 2, C)
    d0[...] = _pack_pair(dr[:, 0, :], dr[:, 1, :])


def _tables_body(feats, w2t, w3t, sel, t0, t1):
    f = feats[...]
    w2 = jnp.dot(f, w2t[...], preferred_element_type=jnp.float32)
    w3 = jnp.dot(f, w3t[...], preferred_element_type=jnp.float32)
    t0[...] = jnp.concatenate([jnp.dot(w2, sel[0]), jnp.dot(w3, sel[0])], 1)
    t1[...] = jnp.concatenate([jnp.dot(w2, sel[1]), jnp.dot(w3, sel[1])], 1)


def _tc_tables(feats, w2t, w3t, sel):
    return pl.pallas_call(
        _tables_body,
        out_shape=[jax.ShapeDtypeStruct((N, C), jnp.float32),
                   jax.ShapeDtypeStruct((N, C), jnp.float32)],
    )(feats, w2t, w3t, sel)


def _tc_prep(hood_p, dw1t, db1, g1, b1, dw2t, db2, g2, b2, jm):
    def full(shape):
        return pl.BlockSpec(shape, lambda i: (0,) * len(shape))
    return pl.pallas_call(
        _prep_body,
        grid=(E // BE,),
        in_specs=[
            pl.BlockSpec((BE, 8), lambda i: (i, 0)),
            full((8, C)), full((1, C)), full((1, C)),
            full((1, C)), full((C, C)), full((1, C)), full((1, C)),
            full((1, C)),
            full((C, C)),
        ],
        out_specs=[
            pl.BlockSpec((BE // 2, C), lambda i: (i, 0)),
        ],
        out_shape=[
            jax.ShapeDtypeStruct((E // 2, C), jnp.int32),
        ],
    )(hood_p, dw1t, db1, g1, b1, dw2t, db2, g2, b2, jm)


def _make_sc_kernel():
    mesh = plsc.VectorSubcoreMesh(core_axis_name="c", subcore_axis_name="s")

    @functools.partial(
        pl.kernel,
        out_type=[jax.ShapeDtypeStruct((N, H), jnp.float32),
                  jax.ShapeDtypeStruct((N, H), jnp.float32)],
        mesh=mesh,
        scratch_types=[
            pltpu.VMEM((CH,), jnp.int32),
            pltpu.VMEM((CH,), jnp.int32),
            pltpu.VMEM((CH,), jnp.int32),
            pltpu.VMEM((CH,), jnp.int32),
            pltpu.VMEM((CH // 2, C), jnp.int32),
            pltpu.VMEM((CH // 2, C), jnp.int32),
            pltpu.VMEM((CH, C), jnp.float32),
            pltpu.VMEM((CH, C), jnp.float32),
            pltpu.VMEM((CH, C), jnp.float32),
            pltpu.VMEM((CH, C), jnp.float32),
            pltpu.VMEM((FH, C), jnp.float32),
            pltpu.VMEM((FH, H), jnp.float32),
            pltpu.VMEM_SHARED((N, C), jnp.float32),
            pltpu.SemaphoreType.DMA,
            pltpu.SemaphoreType.DMA,
            pltpu.SemaphoreType.DMA,
            pltpu.SemaphoreType.DMA,
            pltpu.SemaphoreType.DMA,
            pltpu.SemaphoreType.DMA,
        ],
    )
    def sc_kernel(source, target, dd, t0, t1, out0, out1,
                  idx_s0, idx_s1, idx_t0, idx_t1,
                  dbuf0, dbuf1, rows0, rows1, obuf0, obuf1, fbuf, ob2,
                  acc, semA0, semA1, semB0, semB1, semS0, semS1):
        c = lax.axis_index("c")
        s = lax.axis_index("s")
        bufs = [
            dict(idx_s=idx_s0, idx_t=idx_t0, dbuf=dbuf0,
                 rows=rows0, obuf=obuf0, semA=semA0, semB=semB0, semS=semS0),
            dict(idx_s=idx_s1, idx_t=idx_t1, dbuf=dbuf1,
                 rows=rows1, obuf=obuf1, semA=semA1, semB=semB1, semS=semS1),
        ]

        @plsc.parallel_loop(0, FH, unroll=2)
        def _zero(r):
            for j in range(C // 16):
                fbuf[r, pl.ds(j * 16, 16)] = jnp.zeros((16,), jnp.float32)
        for k in range(-(-NFC // NTILES)):
            m = s + NTILES * k
            if NTILES * (k + 1) <= NFC:
                pltpu.sync_copy(fbuf, acc.at[pl.ds(m * FH, FH)])
            else:
                @pl.when(m < NFC)
                def _():
                    pltpu.sync_copy(fbuf, acc.at[pl.ds(m * FH, FH)])
        plsc.subcore_barrier()

        mask_hi = jnp.full((16,), -65536, jnp.int32)

        def lo16(w):
            return lax.bitcast_convert_type(lax.shift_left(w, 16), jnp.float32)

        def hi16(w):
            return lax.bitcast_convert_type(jnp.bitwise_and(w, mask_hi),
                                            jnp.float32)

        def _main(t_ref, col0):
            def start_fetch(g, B):
                base = s * PT + g * CH
                half = s * (PT // 2) + g * (CH // 2)
                pltpu.async_copy(source.at[pl.ds(base, CH)], B["idx_s"],
                                 B["semA"])
                pltpu.async_copy(target.at[pl.ds(base, CH)], B["idx_t"],
                                 B["semA"])
                pltpu.async_copy(dd.at[pl.ds(half, CH // 2)], B["dbuf"],
                                 B["semA"])

            def wait_fetch(g, B):
                base = s * PT + g * CH
                half = s * (PT // 2) + g * (CH // 2)
                pltpu.make_async_copy(source.at[pl.ds(base, CH)], B["idx_s"],
                                      B["semA"]).wait()
                pltpu.make_async_copy(target.at[pl.ds(base, CH)], B["idx_t"],
                                      B["semA"]).wait()
                pltpu.make_async_copy(dd.at[pl.ds(half, CH // 2)],
                                      B["dbuf"], B["semA"]).wait()

            def wait_scatter(OB):
                pltpu.make_async_copy(
                    OB["obuf"], acc.at[OB["idx_t"]], OB["semS"]).wait()

            def section(p, g, B, OB, prev_scatter_conditional):
                wait_fetch(g, B)
                pltpu.async_copy(t_ref.at[B["idx_s"]], B["rows"], B["semB"])

                if prev_scatter_conditional:
                    @pl.when(p > 0)
                    def _():
                        wait_scatter(OB)
                else:
                    wait_scatter(OB)

                @pl.when(g + 1 < NCHUNK)
                def _():
                    start_fetch(g + 1, OB)

                pltpu.make_async_copy(
                    t_ref.at[B["idx_s"]], B["rows"], B["semB"]).wait()

                dbuf, rows, obuf = B["dbuf"], B["rows"], B["obuf"]

                @plsc.parallel_loop(0, CH // 2, unroll=1)
                def _pairs(i2):
                    dw = [dbuf[i2, pl.ds(col0 + 16 * m, 16)] for m in range(4)]
                    for h, ex in ((lo16, 0), (hi16, 1)):
                        i = 2 * i2 + ex
                        for j in range(4):
                            dv = h(dw[j])
                            w2 = rows[i, pl.ds(j * 16, 16)]
                            w3 = rows[i, pl.ds(H + j * 16, 16)]
                            e = jnp.exp(dv - w2)
                            obuf[i, pl.ds(j * 16, 16)] = e
                            obuf[i, pl.ds(H + j * 16, 16)] = e * (w3 + dv)

                pltpu.async_copy(B["obuf"], acc.at[B["idx_t"]], B["semS"],
                                 add=True)

            start_fetch(0, bufs[0])

            def pair_sections(p, carry):
                section(p, 2 * p, bufs[0], bufs[1], True)
                section(p, 2 * p + 1, bufs[1], bufs[0], False)
                return carry
            lax.fori_loop(0, NCHUNK // 2, pair_sections, 0)

            section(NCHUNK // 2, NCHUNK - 1, bufs[0], bufs[1], False)
            wait_scatter(bufs[0])

        @pl.when(c == 0)
        def _():
            _main(t0, 0)

        @pl.when(c == 1)
        def _():
            _main(t1, H)

        plsc.subcore_barrier()

        def _flush(out_ref):
            def flush_one(m):
                r0 = m * FH
                pltpu.sync_copy(acc.at[pl.ds(r0, FH)], fbuf)

                @plsc.parallel_loop(0, FH, unroll=2)
                def _row(i):
                    for j in range(H // 16):
                        den = fbuf[i, pl.ds(j * 16, 16)]
                        num = fbuf[i, pl.ds(H + j * 16, 16)]
                        ob2[i, pl.ds(j * 16, 16)] = jnp.where(
                            den != 0.0, num / den, 0.0)
                pltpu.sync_copy(ob2, out_ref.at[pl.ds(r0, FH)])

            for k in range(-(-NFC // NTILES)):
                m = s + NTILES * k
                if NTILES * (k + 1) <= NFC:
                    flush_one(m)
                else:
                    @pl.when(m < NFC)
                    def _():
                        flush_one(m)

        @pl.when(c == 0)
        def _():
            _flush(out0)

        @pl.when(c == 1)
        def _():
            _flush(out1)

    return sc_kernel


_SC_KERNEL = None


def kernel(source, target, features, hood_coords, dw1, db1, ln1_g, ln1_b,
           dw2, db2, ln2_g, ln2_b, W1, W2, W3):
    del W1
    global _SC_KERNEL
    if _SC_KERNEL is None:
        _SC_KERNEL = _make_sc_kernel()
    hood_p = jnp.zeros((E, 8), jnp.float32).at[:, :3].set(
        hood_coords.astype(jnp.float32))
    dw1t = jnp.zeros((8, C), jnp.float32).at[:3, :].set(dw1.T)

    def row(v):
        return v.reshape(1, C).astype(jnp.float32)

    jm = jnp.full((C, C), 1.0 / C, jnp.float32)
    eye = jnp.eye(C, dtype=jnp.float32)
    sel = jnp.stack([eye[:, :H], eye[:, H:]])
    t0, t1 = _tc_tables(features, W2.T, W3.T, sel)
    dd, = _tc_prep(
        hood_p, dw1t, row(db1), row(ln1_g), row(ln1_b),
        dw2.T, row(db2), row(ln2_g), row(ln2_b), jm)
    o0, o1 = _SC_KERNEL(source.astype(jnp.int32), target.astype(jnp.int32),
                        dd, t0, t1)
    return jnp.concatenate([o0, o1], axis=1)
